# baseline (speedup 1.0000x reference)
import functools

import jax
import jax.numpy as jnp
from jax import lax
from jax.experimental import pallas as pl
from jax.experimental.pallas import tpu as pltpu
from jax.experimental.pallas import tpu_sc as plsc

_NUM_NODES = 1000000
_EMBED_DIM = 64
_BATCH = 16384

_NC = 2
_NS = 16
_NW = _NC * _NS
_B_PER_W = _BATCH // _NW
_CHUNK = 128
_N_CHUNKS = _B_PER_W // _CHUNK

_mesh = plsc.VectorSubcoreMesh(core_axis_name="c", subcore_axis_name="s")


@functools.partial(
    pl.kernel,
    mesh=_mesh,
    out_type=jax.ShapeDtypeStruct((_BATCH, _EMBED_DIM), jnp.float32),
    compiler_params=pltpu.CompilerParams(use_tc_tiling_on_sc=False),
    scratch_types=[
        pltpu.VMEM((_N_CHUNKS, _CHUNK), jnp.int32),
        pltpu.VMEM((_B_PER_W, _EMBED_DIM), jnp.float32),
        pltpu.SemaphoreType.DMA,
    ],
)
def _gather_kernel(idx_hbm, table_hbm, out_hbm, idx_v, rows_v, sem):
    wid = lax.axis_index("s") * _NC + lax.axis_index("c")
    base = wid * _B_PER_W
    pltpu.sync_copy(idx_hbm.at[wid], idx_v)
    copies = []
    for j in range(_N_CHUNKS):
        copies.append(
            pltpu.async_copy(
                table_hbm.at[idx_v.at[j]],
                rows_v.at[pl.ds(j * _CHUNK, _CHUNK)],
                sem,
            )
        )
    for c in copies:
        c.wait()
    pltpu.sync_copy(rows_v, out_hbm.at[pl.ds(base, _B_PER_W)])


def kernel(batch, table):
    idx = batch.reshape(_NW, _N_CHUNKS, _CHUNK)
    return _gather_kernel(idx, table)


# speedup vs baseline: 1.7380x; 1.7380x over previous
import functools

import jax
import jax.numpy as jnp
from jax import lax
from jax.experimental import pallas as pl
from jax.experimental.pallas import tpu as pltpu
from jax.experimental.pallas import tpu_sc as plsc

_NUM_NODES = 1000000
_EMBED_DIM = 64
_BATCH = 16384

_NC = 2
_NS = 16
_NW = _NC * _NS
_B_PER_W = _BATCH // _NW
_UNROLL = 16
_N_ITERS = _B_PER_W // _UNROLL

_mesh = plsc.VectorSubcoreMesh(core_axis_name="c", subcore_axis_name="s")


@functools.partial(
    pl.kernel,
    mesh=_mesh,
    out_type=jax.ShapeDtypeStruct((_BATCH, _EMBED_DIM), jnp.float32),
    scratch_types=[
        pltpu.VMEM((_B_PER_W,), jnp.int32),
        pltpu.VMEM((_B_PER_W, _EMBED_DIM), jnp.float32),
        pltpu.SemaphoreType.DMA,
    ],
)
def _gather_kernel(idx_hbm, table_hbm, out_hbm, idx_v, rows_v, sem):
    wid = lax.axis_index("s") * _NC + lax.axis_index("c")
    base = wid * _B_PER_W
    pltpu.sync_copy(idx_hbm.at[pl.ds(base, _B_PER_W)], idx_v)

    @pl.loop(0, _N_ITERS)
    def _issue(i):
        vec = idx_v[pl.ds(i * _UNROLL, _UNROLL)]
        for j in range(_UNROLL):
            r = i * _UNROLL + j
            n = vec[j]
            pltpu.async_copy(
                table_hbm.at[pl.ds(n, 1), :],
                rows_v.at[pl.ds(r, 1), :],
                sem,
            )

    pltpu.make_async_copy(
        table_hbm.at[pl.ds(0, _B_PER_W), :], rows_v, sem
    ).wait()

    pltpu.sync_copy(rows_v, out_hbm.at[pl.ds(base, _B_PER_W)])


def kernel(batch, table):
    return _gather_kernel(batch, table)


# speedup vs baseline: 1.9866x; 1.1430x over previous
import functools

import jax
import jax.numpy as jnp
from jax import lax
from jax.experimental import pallas as pl
from jax.experimental.pallas import tpu as pltpu
from jax.experimental.pallas import tpu_sc as plsc

_NUM_NODES = 1000000
_EMBED_DIM = 64
_BATCH = 16384

_NC = 2
_NS = 16
_NW = _NC * _NS
_B_PER_W = _BATCH // _NW
_L = 16
_G = 128
_NBUF = 4

_mesh = plsc.VectorSubcoreMesh(core_axis_name="c", subcore_axis_name="s")


@functools.partial(
    pl.kernel,
    mesh=_mesh,
    out_type=jax.ShapeDtypeStruct((_BATCH * _EMBED_DIM,), jnp.float32),
    compiler_params=pltpu.CompilerParams(needs_layout_passes=False),
    scratch_types=[
        pltpu.VMEM((_B_PER_W,), jnp.int32),
        pltpu.VMEM((_NBUF, _EMBED_DIM, _G), jnp.float32),
        pltpu.VMEM((_B_PER_W * _EMBED_DIM,), jnp.float32),
        pltpu.SemaphoreType.DMA,
    ],
)
def _gather_kernel(idx_hbm, table_hbm, out_hbm, idx_v, stage, outb, sem):
    wid = lax.axis_index("s") * _NC + lax.axis_index("c")
    base = wid * _B_PER_W
    pltpu.sync_copy(idx_hbm.at[pl.ds(base, _B_PER_W)], idx_v)

    kvecs = [lax.iota(jnp.int32, _L) + (g * _L) for g in range(_EMBED_DIM // _L)]

    @pl.loop(0, _B_PER_W // _L)
    def _group(g):
        vec = idx_v[pl.ds(g * _L, _L)]
        qvec = lax.shift_right_logical(vec, 7) * _G
        cvec = lax.bitwise_and(vec, _G - 1)
        for b in range(_L // _NBUF):
            for j in range(_NBUF):
                q = pl.multiple_of(qvec[b * _NBUF + j], _G)
                pltpu.async_copy(
                    table_hbm.at[:, pl.ds(q, _G)], stage.at[j], sem
                )
            for j in range(_NBUF):
                pltpu.make_async_copy(
                    table_hbm.at[:, pl.ds(0, _G)], stage.at[j], sem
                ).wait()
            for j in range(_NBUF):
                r = g * _L + b * _NBUF + j
                c = cvec[b * _NBUF + j]
                for kg, kvec in enumerate(kvecs):
                    vals = plsc.load_gather(
                        stage.at[j], [kvec, jnp.full((_L,), c, jnp.int32)]
                    )
                    outb[pl.ds(r * _EMBED_DIM + kg * _L, _L)] = vals

    pltpu.sync_copy(outb, out_hbm.at[pl.ds(base * _EMBED_DIM, _B_PER_W * _EMBED_DIM)])


def kernel(batch, table):
    flat = _gather_kernel(batch, table.T)
    return flat.reshape(_BATCH, _EMBED_DIM)


# speedup vs baseline: 2.4259x; 1.2211x over previous
import functools

import jax
import jax.numpy as jnp
from jax import lax
from jax.experimental import pallas as pl
from jax.experimental.pallas import tpu as pltpu
from jax.experimental.pallas import tpu_sc as plsc

_NUM_NODES = 1000000
_EMBED_DIM = 64
_BATCH = 16384

_NC = 2
_NS = 16
_NW = _NC * _NS
_B_PER_W = _BATCH // _NW
_L = 16
_G = 128
_NBUF = 4
_N_BATCH = _B_PER_W // _NBUF

_mesh = plsc.VectorSubcoreMesh(core_axis_name="c", subcore_axis_name="s")


@functools.partial(
    pl.kernel,
    mesh=_mesh,
    out_type=jax.ShapeDtypeStruct((_BATCH * _EMBED_DIM,), jnp.float32),
    compiler_params=pltpu.CompilerParams(needs_layout_passes=False),
    scratch_types=[
        pltpu.VMEM((_B_PER_W + _L,), jnp.int32),
        pltpu.VMEM((2 * _NBUF, _EMBED_DIM, _G), jnp.float32),
        pltpu.VMEM((_B_PER_W * _EMBED_DIM,), jnp.float32),
        pltpu.SemaphoreType.DMA,
        pltpu.SemaphoreType.DMA,
    ],
)
def _gather_kernel(idx_hbm, table_hbm, out_hbm, idx_v, stage, outb,
                   sem_a, sem_b):
    wid = lax.axis_index("s") * _NC + lax.axis_index("c")
    base = wid * _B_PER_W
    pltpu.sync_copy(idx_hbm.at[pl.ds(base, _B_PER_W)],
                    idx_v.at[pl.ds(0, _B_PER_W)])

    kvecs = [lax.iota(jnp.int32, _L) + (g * _L) for g in range(_EMBED_DIM // _L)]

    def fire(t, half, sem):
        vec = idx_v[pl.ds(t * _NBUF, _L)]
        for j in range(_NBUF):
            q = pl.multiple_of(
                lax.shift_right_logical(vec[j], 7) * _G, _G
            )
            pltpu.async_copy(
                table_hbm.at[:, pl.ds(q, _G)],
                stage.at[half * _NBUF + j],
                sem,
            )

    def drain_extract(t, half, sem):
        for j in range(_NBUF):
            pltpu.make_async_copy(
                table_hbm.at[:, pl.ds(0, _G)],
                stage.at[half * _NBUF + j],
                sem,
            ).wait()
        vec = idx_v[pl.ds(t * _NBUF, _L)]
        cvec = lax.bitwise_and(vec, _G - 1)
        for j in range(_NBUF):
            r = t * _NBUF + j
            c = cvec[j]
            for kg, kvec in enumerate(kvecs):
                vals = plsc.load_gather(
                    stage.at[half * _NBUF + j],
                    [kvec, jnp.full((_L,), c, jnp.int32)],
                )
                outb[pl.ds(r * _EMBED_DIM + kg * _L, _L)] = vals

    fire(0, 0, sem_a)
    fire(1, 1, sem_b)

    @pl.loop(0, _N_BATCH // 2)
    def _pair(u):
        t0 = u * 2

        drain_extract(t0, 0, sem_a)

        @pl.when(t0 + 2 < _N_BATCH)
        def _():
            fire(t0 + 2, 0, sem_a)

        drain_extract(t0 + 1, 1, sem_b)

        @pl.when(t0 + 3 < _N_BATCH)
        def _():
            fire(t0 + 3, 1, sem_b)

    pltpu.sync_copy(
        outb, out_hbm.at[pl.ds(base * _EMBED_DIM, _B_PER_W * _EMBED_DIM)]
    )


def kernel(batch, table):
    flat = _gather_kernel(batch, table.T)
    return flat.reshape(_BATCH, _EMBED_DIM)
